# baseline (speedup 1.0000x reference)
import jax
import jax.numpy as jnp
from jax.experimental import pallas as pl

DIM = 512
N_GROUPS = 8
N_CODES = 1024
SUB = DIM // N_GROUPS
T_TILE = 512


def _rvq_kernel(x_ref, w1_ref, w2_ref, qout_ref, idx_ref, loss_ref):
    b = pl.program_id(0)
    i = pl.program_id(1)

    @pl.when((b == 0) & (i == 0))
    def _init():
        loss_ref[...] = jnp.zeros_like(loss_ref)

    tt = x_ref.shape[2]
    step_sum = jnp.float32(0.0)
    for g in range(N_GROUPS):
        xg = x_ref[0, g * SUB:(g + 1) * SUB, :]
        r = xg
        zq_acc = None
        for s, w_ref in enumerate((w1_ref, w2_ref)):
            wg = w_ref[g]
            wsq = jnp.sum(wg * wg, axis=1, keepdims=True)
            wneg2 = -(wg + wg)
            rsq = jnp.sum(r * r, axis=0, keepdims=True)
            pneg = jax.lax.dot_general(
                wneg2, r, (((1,), (0,)), ((), ())),
                preferred_element_type=jnp.float32)
            iota8 = jax.lax.broadcasted_iota(jnp.int32, (8, 1), 0)
            run_val = None
            for c in range(N_CODES // 8):
                dch = (rsq + wsq[c * 8:(c + 1) * 8, :]) \
                    + pneg[c * 8:(c + 1) * 8, :]
                ic = iota8 + (c * 8)
                if run_val is None:
                    run_val = dch
                    run_idx = jnp.broadcast_to(ic, dch.shape)
                else:
                    lt = dch < run_val
                    run_val = jnp.where(lt, dch, run_val)
                    run_idx = jnp.where(lt, ic, run_idx)
            dmin = jnp.min(run_val, axis=0, keepdims=True)
            idx = jnp.min(jnp.where(run_val == dmin, run_idx, N_CODES),
                          axis=0)
            iota_c = jax.lax.broadcasted_iota(jnp.int32, (N_CODES, 1), 0)
            oh = jnp.where(iota_c == idx[None, :],
                           jnp.float32(1.0), jnp.float32(0.0))
            zq = jax.lax.dot_general(
                wg, oh, (((0,), (0,)), ((), ())),
                preferred_element_type=jnp.float32)
            t = zq - r
            step_sum = step_sum + jnp.sum(t * t)
            zq_st = r + t
            r = r - zq_st
            zq_acc = zq_st if zq_acc is None else zq_acc + zq_st
            idx_ref[s, g, :] = idx
        qout_ref[0, g * SUB:(g + 1) * SUB, :] = zq_acc

    loss_ref[...] = loss_ref[...] + step_sum


def kernel(x, W1, W2):
    B, C, T = x.shape
    nt = T // T_TILE
    grid = (B, nt)
    qout, idx, loss = pl.pallas_call(
        _rvq_kernel,
        grid=grid,
        in_specs=[
            pl.BlockSpec((1, DIM, T_TILE), lambda b, i: (b, 0, i)),
            pl.BlockSpec((N_GROUPS, N_CODES, SUB), lambda b, i: (0, 0, 0)),
            pl.BlockSpec((N_GROUPS, N_CODES, SUB), lambda b, i: (0, 0, 0)),
        ],
        out_specs=[
            pl.BlockSpec((1, DIM, T_TILE), lambda b, i: (b, 0, i)),
            pl.BlockSpec((2, N_GROUPS, T_TILE),
                         lambda b, i, _nt=nt: (0, 0, b * _nt + i)),
            pl.BlockSpec((1, 1), lambda b, i: (0, 0)),
        ],
        out_shape=[
            jax.ShapeDtypeStruct((B, DIM, T), jnp.float32),
            jax.ShapeDtypeStruct((2, N_GROUPS, B * T), jnp.int32),
            jax.ShapeDtypeStruct((1, 1), jnp.float32),
        ],
    )(x, W1, W2)
    numel = B * C * T
    total_loss = loss[0, 0] * (1.25 / (2.0 * numel))
    return (qout, total_loss, idx)


# speedup vs baseline: 1.2830x; 1.2830x over previous
import jax
import jax.numpy as jnp
from jax.experimental import pallas as pl
from jax.experimental.pallas import tpu as pltpu

DIM = 512
N_GROUPS = 8
N_CODES = 1024
SUB = DIM // N_GROUPS
T_TILE = 1024


def _rvq_kernel(x_ref, w1_ref, w2_ref, qout_ref, idx_ref, loss_ref):
    tt = x_ref.shape[2]
    step_sum = jnp.float32(0.0)
    for g in range(N_GROUPS):
        xg = x_ref[0, g * SUB:(g + 1) * SUB, :]
        r = xg
        zq_acc = None
        for s, w_ref in enumerate((w1_ref, w2_ref)):
            wg = w_ref[g]
            wsq = jnp.sum(wg * wg, axis=1, keepdims=True)
            wneg2 = -(wg + wg)
            rsq = jnp.sum(r * r, axis=0, keepdims=True)
            pneg = jax.lax.dot_general(
                wneg2, r, (((1,), (0,)), ((), ())),
                preferred_element_type=jnp.float32)
            iota8 = jax.lax.broadcasted_iota(jnp.int32, (8, 1), 0)
            run_val = None
            for c in range(N_CODES // 8):
                dch = (rsq + wsq[c * 8:(c + 1) * 8, :]) \
                    + pneg[c * 8:(c + 1) * 8, :]
                ic = iota8 + (c * 8)
                if run_val is None:
                    run_val = dch
                    run_idx = jnp.broadcast_to(ic, dch.shape)
                else:
                    lt = dch < run_val
                    run_val = jnp.where(lt, dch, run_val)
                    run_idx = jnp.where(lt, ic, run_idx)
            dmin = jnp.min(run_val, axis=0, keepdims=True)
            idx = jnp.min(jnp.where(run_val == dmin, run_idx, N_CODES),
                          axis=0)
            iota_c = jax.lax.broadcasted_iota(jnp.int32, (N_CODES, 1), 0)
            oh = jnp.where(iota_c == idx[None, :],
                           jnp.float32(1.0), jnp.float32(0.0))
            zq = jax.lax.dot_general(
                wg, oh, (((0,), (0,)), ((), ())),
                preferred_element_type=jnp.float32)
            t = zq - r
            step_sum = step_sum + jnp.sum(t * t)
            zq_st = r + t
            r = r - zq_st
            zq_acc = zq_st if zq_acc is None else zq_acc + zq_st
            idx_ref[s, g, :] = idx
        qout_ref[0, g * SUB:(g + 1) * SUB, :] = zq_acc

    loss_ref[...] = jnp.full(loss_ref.shape, step_sum, jnp.float32)


def kernel(x, W1, W2):
    B, C, T = x.shape
    nt = T // T_TILE
    grid = (B, nt)
    qout, idx, loss = pl.pallas_call(
        _rvq_kernel,
        grid=grid,
        in_specs=[
            pl.BlockSpec((1, DIM, T_TILE), lambda b, i: (b, 0, i)),
            pl.BlockSpec((N_GROUPS, N_CODES, SUB), lambda b, i: (0, 0, 0)),
            pl.BlockSpec((N_GROUPS, N_CODES, SUB), lambda b, i: (0, 0, 0)),
        ],
        out_specs=[
            pl.BlockSpec((1, DIM, T_TILE), lambda b, i: (b, 0, i)),
            pl.BlockSpec((2, N_GROUPS, T_TILE),
                         lambda b, i, _nt=nt: (0, 0, b * _nt + i)),
            pl.BlockSpec((8, 128), lambda b, i, _nt=nt: (b * _nt + i, 0)),
        ],
        out_shape=[
            jax.ShapeDtypeStruct((B, DIM, T), jnp.float32),
            jax.ShapeDtypeStruct((2, N_GROUPS, B * T), jnp.int32),
            jax.ShapeDtypeStruct((B * nt * 8, 128), jnp.float32),
        ],
        compiler_params=pltpu.CompilerParams(
            dimension_semantics=("parallel", "parallel")),
    )(x, W1, W2)
    numel = B * C * T
    total_loss = jnp.sum(loss[::8, 0]) * (1.25 / (2.0 * numel))
    return (qout, total_loss, idx)
